# baseline (speedup 1.0000x reference)
import jax
import jax.numpy as jnp
from jax import lax
from jax.experimental import pallas as pl
from jax.experimental.pallas import tpu as pltpu
from jax.experimental.pallas import tpu_sc as plsc

N_USERS = 30000
N_ITEMS = 20000
N_NODES = 50000
D = 64
DH = 32
LAYERS = 3
E = 800000

NC, NS = 2, 16
HALF = 25600
N_PAD = NC * HALF
ACC_ROWS = 25728
TRASH = 25600
ROWS_PER_TILE = HALF // NS
BCHUNK = 160
BSTEPS = ROWS_PER_TILE // BCHUNK
E_PAD = 819200
EROWS = E_PAD // 128
EROWS_PER_TILE = EROWS // NS
STRIP_ROWS = 16
NSTRIPS = EROWS_PER_TILE // STRIP_ROWS
FROWS = STRIP_ROWS + 1
ZROWS_T = HALF // NS
G = 4

_MESH = plsc.VectorSubcoreMesh(core_axis_name="c", subcore_axis_name="s")


def _popcnt(m):
    r = plsc.all_reduce_population_count(m)
    if getattr(r, "ndim", 0):
        r = r[0]
    return r


def _zero_rows(rows_v, n):
    def z0body(i, _):
        for k in range(DH // 16):
            rows_v[i, pl.ds(k * 16, 16)] = jnp.zeros((16,), jnp.float32)
        return 0

    lax.fori_loop(0, n, z0body, 0)


def _zero_acc(acc, rows_v, s):
    z0 = s * ZROWS_T

    def zbody(i, _):
        pltpu.sync_copy(rows_v.at[pl.ds(0, 128)],
                        acc.at[pl.ds(z0 + i * 128, 128)])
        return 0

    lax.fori_loop(0, ZROWS_T // 128, zbody, 0)
    rem = ZROWS_T - (ZROWS_T // 128) * 128
    if rem:
        pltpu.sync_copy(rows_v.at[pl.ds(0, rem)],
                        acc.at[pl.ds(z0 + ZROWS_T - rem, rem)])


def _filter_strip(src_v, dst_v, srcf, ldstf, base):
    iot = lax.iota(jnp.int32, 16)

    def fbody(i, cur):
        j = i >> 3
        k = i & 7
        sl = pl.ds(k * 16, 16)
        vd = dst_v[j, sl]
        vs = src_v[j, sl]
        l = vd - base
        m = (l >= 0) & (l < HALF)
        m32 = jnp.where(m, 1, 0)
        rank = plsc.cumsum(m32) - m32
        a = cur + rank
        plsc.store_scatter(srcf, [a >> 7, a & 127], vs, mask=m)
        plsc.store_scatter(ldstf, [a >> 7, a & 127], l, mask=m)
        return cur + _popcnt(m)

    cur = lax.fori_loop(0, STRIP_ROWS * 8, fbody, 0)
    nb = (cur + 127) >> 7
    lim = nb * 128

    def pbody(k, _):
        a = cur + k * 16 + iot
        m = a < lim
        psrc = 50000 + (a & 1023)
        plsc.store_scatter(srcf, [a >> 7, a & 127], psrc, mask=m)
        plsc.store_scatter(ldstf, [a >> 7, a & 127], TRASH + (a & 127),
                           mask=m)
        return 0

    lax.fori_loop(0, 8, pbody, 0)
    return nb


def _deg_body(src_hbm, dst_hbm, deg_hbm, dacc, src_v, dst_v, srcf, ldstf,
              ones_v, degb, sem_s):
    c = lax.axis_index("c")
    s = lax.axis_index("s")

    def initv(i, _):
        ones_v[pl.ds(i * 16, 16)] = jnp.ones((16,), jnp.float32)
        degb[pl.ds(i * 16, 16)] = jnp.zeros((16,), jnp.float32)
        return 0

    lax.fori_loop(0, 8, initv, 0)

    z0 = s * ZROWS_T

    def zbody(i, _):
        pltpu.sync_copy(degb.at[pl.ds(0, 128)],
                        dacc.at[pl.ds(z0 + i * 128, 128)])
        return 0

    lax.fori_loop(0, ZROWS_T // 128, zbody, 0)
    rem = ZROWS_T - (ZROWS_T // 128) * 128
    if rem:
        pltpu.sync_copy(degb.at[pl.ds(0, rem)],
                        dacc.at[pl.ds(z0 + ZROWS_T - rem, rem)])
    plsc.subcore_barrier()

    base = c * HALF

    def strip(st, _):
        row0 = s * EROWS_PER_TILE + st * STRIP_ROWS
        pltpu.sync_copy(src_hbm.at[pl.ds(row0, STRIP_ROWS)], src_v)
        pltpu.sync_copy(dst_hbm.at[pl.ds(row0, STRIP_ROWS)], dst_v)
        nb = _filter_strip(src_v, dst_v, srcf, ldstf, base)

        def bloop(b, _):
            pltpu.async_copy(ones_v, dacc.at[ldstf.at[b]], sem_s, add=True)
            return 0

        lax.fori_loop(0, nb, bloop, 0)

        def bdrain(b, _):
            pltpu.make_async_copy(ones_v, dacc.at[ldstf.at[b]], sem_s).wait()
            return 0

        lax.fori_loop(0, nb, bdrain, 0)
        return 0

    lax.fori_loop(0, NSTRIPS, strip, 0)
    plsc.subcore_barrier()

    lb = s * ROWS_PER_TILE
    pltpu.sync_copy(dacc.at[pl.ds(lb, ROWS_PER_TILE)], degb)
    pltpu.sync_copy(degb, deg_hbm.at[pl.ds(c * HALF + lb, ROWS_PER_TILE)])


def _pass(src_hbm, dst_hbm, yin, de_hbm, d2_hbm, tin, yout, tout,
          acc, src_v, dst_v, srcf, ldstf, rows_v, sem_g, sem_s, c, s):
    base = c * HALF
    _zero_acc(acc, rows_v, s)
    plsc.subcore_barrier()

    def strip(st, _):
        row0 = s * EROWS_PER_TILE + st * STRIP_ROWS
        pltpu.sync_copy(src_hbm.at[pl.ds(row0, STRIP_ROWS)], src_v)
        pltpu.sync_copy(dst_hbm.at[pl.ds(row0, STRIP_ROWS)], dst_v)
        nb = _filter_strip(src_v, dst_v, srcf, ldstf, base)
        ngroups = (nb + G - 1) >> 2

        def fire_g(b, half):
            pltpu.async_copy(yin.at[srcf.at[b]],
                             rows_v.at[pl.ds(half * (G * 128)
                                             + (b & 3) * 128, 128)], sem_g)

        def wait_g(b, half):
            pltpu.make_async_copy(yin.at[srcf.at[b]],
                                  rows_v.at[pl.ds(half * (G * 128)
                                                  + (b & 3) * 128, 128)],
                                  sem_g).wait()

        def fire_s(b, half):
            pltpu.async_copy(rows_v.at[pl.ds(half * (G * 128)
                                             + (b & 3) * 128, 128)],
                             acc.at[ldstf.at[b]], sem_s, add=True)

        def wait_s(b, half):
            pltpu.make_async_copy(rows_v.at[pl.ds(half * (G * 128)
                                                  + (b & 3) * 128, 128)],
                                  acc.at[ldstf.at[b]], sem_s).wait()

        for u in range(G):
            @pl.when(u < nb)
            def _(u=u):
                fire_g(u, 0)

        def gloop(p, _):
            half = p & 1
            nxt = 1 - half
            for u in range(G):
                b = p * G + u

                @pl.when(b < nb)
                def _(b=b, half=half):
                    wait_g(b, half)

            for u in range(G):
                b = (p - 1) * G + u

                @pl.when(p > 0)
                def _(b=b, nxt=nxt):
                    wait_s(b, nxt)

            for u in range(G):
                b = p * G + u

                @pl.when(b < nb)
                def _(b=b, half=half):
                    fire_s(b, half)

            for u in range(G):
                b = (p + 1) * G + u

                @pl.when(b < nb)
                def _(b=b, nxt=nxt):
                    fire_g(b, nxt)

            return 0

        lax.fori_loop(0, ngroups, gloop, 0)

        def drain(_):
            p = ngroups - 1
            for u in range(G):
                b = p * G + u

                @pl.when(b < nb)
                def _(b=b, p=p):
                    wait_s(b, p & 1)

            return 0

        lax.cond(ngroups > 0, drain, lambda _: 0, 0)
        return 0

    lax.fori_loop(0, NSTRIPS, strip, 0)
    plsc.subcore_barrier()

    lb = s * ROWS_PER_TILE

    def bstep(k, _):
        r0 = lb + k * BCHUNK
        g0 = c * HALF + r0
        pltpu.sync_copy(acc.at[pl.ds(r0, BCHUNK)], rows_v.at[pl.ds(0, BCHUNK)])
        pltpu.sync_copy(de_hbm.at[pl.ds(g0, BCHUNK)],
                        rows_v.at[pl.ds(160, BCHUNK)])
        pltpu.sync_copy(d2_hbm.at[pl.ds(g0, BCHUNK)],
                        rows_v.at[pl.ds(320, BCHUNK)])
        pltpu.sync_copy(tin.at[pl.ds(g0, BCHUNK)],
                        rows_v.at[pl.ds(480, BCHUNK)])

        def rowop(i, _):
            for q in range(DH // 16):
                sl = pl.ds(q * 16, 16)
                a = rows_v[i, sl]
                rows_v[480 + i, sl] = (rows_v[480 + i, sl]
                                       + 0.25 * (rows_v[160 + i, sl] * a))
                rows_v[i, sl] = rows_v[320 + i, sl] * a
            return 0

        lax.fori_loop(0, BCHUNK, rowop, 0)
        pltpu.sync_copy(rows_v.at[pl.ds(480, BCHUNK)],
                        tout.at[pl.ds(g0, BCHUNK)])
        pltpu.sync_copy(rows_v.at[pl.ds(0, BCHUNK)],
                        yout.at[pl.ds(g0, BCHUNK)])
        return 0

    lax.fori_loop(0, BSTEPS, bstep, 0)


def _layer_body(src_hbm, dst_hbm, ylo, yhi, de_hbm, d2_hbm, tlo, thi,
                ylo2, yhi2, tlo2, thi2,
                acc, src_v, dst_v, srcf, ldstf, rows_v, sem_g, sem_s):
    c = lax.axis_index("c")
    s = lax.axis_index("s")
    _zero_rows(rows_v, 128)
    _pass(src_hbm, dst_hbm, ylo, de_hbm, d2_hbm, tlo, ylo2, tlo2,
          acc, src_v, dst_v, srcf, ldstf, rows_v, sem_g, sem_s, c, s)
    plsc.subcore_barrier()
    _zero_rows(rows_v, 128)
    _pass(src_hbm, dst_hbm, yhi, de_hbm, d2_hbm, thi, yhi2, thi2,
          acc, src_v, dst_v, srcf, ldstf, rows_v, sem_g, sem_s, c, s)


def _setup_tc_body(deg_ref, w_ref, de_ref, d2_ref, ylo_ref, yhi_ref,
                   tlo_ref, thi_ref):
    d = deg_ref[:, :]
    dis = jnp.where(d >= 0.5, lax.rsqrt(jnp.maximum(d, 1e-12)), 0.0)
    w = w_ref[:, :]
    de = jnp.broadcast_to(dis, (dis.shape[0], DH))
    de_ref[:, :] = de
    d2_ref[:, :] = de * de
    ylo_ref[:, :] = de * w[:, :DH]
    yhi_ref[:, :] = de * w[:, DH:]
    tlo_ref[:, :] = 0.25 * w[:, :DH]
    thi_ref[:, :] = 0.25 * w[:, DH:]


_SETUP_BR = 512


def _deg_call(src2d, dst2d):
    f = pl.kernel(
        _deg_body,
        out_type=jax.ShapeDtypeStruct((N_PAD,), jnp.float32),
        mesh=_MESH,
        scratch_types=[
            pltpu.VMEM_SHARED((ACC_ROWS,), jnp.float32),
            pltpu.VMEM((STRIP_ROWS, 128), jnp.int32),
            pltpu.VMEM((STRIP_ROWS, 128), jnp.int32),
            pltpu.VMEM((FROWS, 128), jnp.int32),
            pltpu.VMEM((FROWS, 128), jnp.int32),
            pltpu.VMEM((128,), jnp.float32),
            pltpu.VMEM((ROWS_PER_TILE,), jnp.float32),
            pltpu.SemaphoreType.DMA,
        ],
        compiler_params=pltpu.CompilerParams(use_tc_tiling_on_sc=False,
                                             needs_layout_passes=False),
    )
    return f(src2d, dst2d)


def _setup_call(deg, w_pad):
    grid = (N_PAD // _SETUP_BR,)
    bs_deg = pl.BlockSpec((_SETUP_BR, 1), lambda i: (i, 0))
    bs_w = pl.BlockSpec((_SETUP_BR, D), lambda i: (i, 0))
    bs = pl.BlockSpec((_SETUP_BR, DH), lambda i: (i, 0))
    out = jax.ShapeDtypeStruct((N_PAD, DH), jnp.float32)
    return pl.pallas_call(
        _setup_tc_body,
        grid=grid,
        in_specs=[bs_deg, bs_w],
        out_specs=[bs, bs, bs, bs, bs, bs],
        out_shape=[out, out, out, out, out, out],
    )(deg, w_pad)


def _layer_call(src2d, dst2d, ylo, yhi, de, d2, tlo, thi):
    out = jax.ShapeDtypeStruct((N_PAD, DH), jnp.float32)
    f = pl.kernel(
        _layer_body,
        out_type=(out, out, out, out),
        mesh=_MESH,
        scratch_types=[
            pltpu.VMEM_SHARED((ACC_ROWS, DH), jnp.float32),
            pltpu.VMEM((STRIP_ROWS, 128), jnp.int32),
            pltpu.VMEM((STRIP_ROWS, 128), jnp.int32),
            pltpu.VMEM((FROWS, 128), jnp.int32),
            pltpu.VMEM((FROWS, 128), jnp.int32),
            pltpu.VMEM((2 * G * 128, DH), jnp.float32),
            pltpu.SemaphoreType.DMA,
            pltpu.SemaphoreType.DMA,
        ],
        compiler_params=pltpu.CompilerParams(use_tc_tiling_on_sc=False,
                                             needs_layout_passes=False),
    )
    return f(src2d, dst2d, ylo, yhi, de, d2, tlo, thi)


def kernel(edge_index, weight):
    src = edge_index[0]
    dst = edge_index[1]
    pad = jnp.full((E_PAD - E,), 1 << 29, dtype=jnp.int32)
    src2d = jnp.concatenate([src, jnp.zeros((E_PAD - E,), jnp.int32)]
                            ).reshape(EROWS, 128)
    dst2d = jnp.concatenate([dst, pad]).reshape(EROWS, 128)
    w_pad = jnp.zeros((N_PAD, D), jnp.float32).at[:N_NODES].set(weight)

    deg = _deg_call(src2d, dst2d)
    de, d2, ylo, yhi, tlo, thi = _setup_call(deg.reshape(N_PAD, 1), w_pad)
    for _ in range(LAYERS):
        ylo, yhi, tlo, thi = _layer_call(src2d, dst2d, ylo, yhi, de, d2,
                                         tlo, thi)

    t = jnp.concatenate([tlo, thi], axis=1)
    return (t[:N_USERS], t[N_USERS:N_NODES])


# speedup vs baseline: 1.1015x; 1.1015x over previous
import jax
import jax.numpy as jnp
from jax import lax
from jax.experimental import pallas as pl
from jax.experimental.pallas import tpu as pltpu
from jax.experimental.pallas import tpu_sc as plsc

N_USERS = 30000
N_ITEMS = 20000
N_NODES = 50000
D = 64
LAYERS = 3
E = 800000

NC, NS = 2, 16
HALF = 25600
N_PAD = NC * HALF
ACC_ROWS = 25728
TRASH = 25600
ROWS_PER_TILE = HALF // NS
BCHUNK = 64
BSTEPS = ROWS_PER_TILE // BCHUNK
E_PAD = 819200
EROWS = E_PAD // 128
EROWS_PER_TILE = EROWS // NS
STRIP_ROWS = 16
NSTRIPS = EROWS_PER_TILE // STRIP_ROWS
FROWS = STRIP_ROWS + 1
ZROWS_T = HALF // NS

_MESH = plsc.VectorSubcoreMesh(core_axis_name="c", subcore_axis_name="s")


def _popcnt(m):
    r = plsc.all_reduce_population_count(m)
    if getattr(r, "ndim", 0):
        r = r[0]
    return r


def _zero_rows(rows_v, n):
    def z0body(i, _):
        for k in range(4):
            rows_v[i, pl.ds(k * 16, 16)] = jnp.zeros((16,), jnp.float32)
        return 0

    lax.fori_loop(0, n, z0body, 0)


def _zero_acc(acc, rows_v, s):
    z0 = s * ZROWS_T

    def zbody(i, _):
        pltpu.sync_copy(rows_v.at[pl.ds(0, 128)],
                        acc.at[pl.ds(z0 + i * 128, 128)])
        return 0

    lax.fori_loop(0, ZROWS_T // 128, zbody, 0)
    rem = ZROWS_T - (ZROWS_T // 128) * 128
    if rem:
        pltpu.sync_copy(rows_v.at[pl.ds(0, rem)],
                        acc.at[pl.ds(z0 + ZROWS_T - rem, rem)])


def _filter_strip(src_v, dst_v, srcf, ldstf, base):
    iot = lax.iota(jnp.int32, 16)

    def fbody(i, cur):
        j = i >> 3
        k = i & 7
        sl = pl.ds(k * 16, 16)
        vd = dst_v[j, sl]
        vs = src_v[j, sl]
        l = vd - base
        m = (l >= 0) & (l < HALF)
        m32 = jnp.where(m, 1, 0)
        rank = plsc.cumsum(m32) - m32
        a = cur + rank
        plsc.store_scatter(srcf, [a >> 7, a & 127], vs, mask=m)
        plsc.store_scatter(ldstf, [a >> 7, a & 127], l, mask=m)
        return cur + _popcnt(m)

    cur = lax.fori_loop(0, STRIP_ROWS * 8, fbody, 0)
    nb = (cur + 127) >> 7
    lim = nb * 128

    def pbody(k, _):
        a = cur + k * 16 + iot
        m = a < lim
        psrc = 50000 + (a & 1023)
        plsc.store_scatter(srcf, [a >> 7, a & 127], psrc, mask=m)
        plsc.store_scatter(ldstf, [a >> 7, a & 127], TRASH + (a & 127),
                           mask=m)
        return 0

    lax.fori_loop(0, 8, pbody, 0)
    return nb


def _deg_body(src_hbm, dst_hbm, deg_hbm, dacc, src_v, dst_v, srcf, ldstf,
              ones_v, degb, sem_s):
    c = lax.axis_index("c")
    s = lax.axis_index("s")

    def initv(i, _):
        ones_v[pl.ds(i * 16, 16)] = jnp.ones((16,), jnp.float32)
        degb[pl.ds(i * 16, 16)] = jnp.zeros((16,), jnp.float32)
        return 0

    lax.fori_loop(0, 8, initv, 0)

    z0 = s * ZROWS_T

    def zbody(i, _):
        pltpu.sync_copy(degb.at[pl.ds(0, 128)],
                        dacc.at[pl.ds(z0 + i * 128, 128)])
        return 0

    lax.fori_loop(0, ZROWS_T // 128, zbody, 0)
    rem = ZROWS_T - (ZROWS_T // 128) * 128
    if rem:
        pltpu.sync_copy(degb.at[pl.ds(0, rem)],
                        dacc.at[pl.ds(z0 + ZROWS_T - rem, rem)])
    plsc.subcore_barrier()

    base = c * HALF

    def strip(st, _):
        row0 = s * EROWS_PER_TILE + st * STRIP_ROWS
        pltpu.sync_copy(src_hbm.at[pl.ds(row0, STRIP_ROWS)], src_v)
        pltpu.sync_copy(dst_hbm.at[pl.ds(row0, STRIP_ROWS)], dst_v)
        nb = _filter_strip(src_v, dst_v, srcf, ldstf, base)

        def bloop(b, _):
            pltpu.async_copy(ones_v, dacc.at[ldstf.at[b]], sem_s, add=True)
            return 0

        lax.fori_loop(0, nb, bloop, 0)

        def bdrain(b, _):
            pltpu.make_async_copy(ones_v, dacc.at[ldstf.at[b]], sem_s).wait()
            return 0

        lax.fori_loop(0, nb, bdrain, 0)
        return 0

    lax.fori_loop(0, NSTRIPS, strip, 0)
    plsc.subcore_barrier()

    lb = s * ROWS_PER_TILE
    pltpu.sync_copy(dacc.at[pl.ds(lb, ROWS_PER_TILE)], degb)
    pltpu.sync_copy(degb, deg_hbm.at[pl.ds(c * HALF + lb, ROWS_PER_TILE)])


def _layer_body(src_hbm, dst_hbm, y_hbm, de_hbm, d2_hbm, tin_hbm,
                yout_hbm, tout_hbm,
                acc, src_v, dst_v, srcf, ldstf, rows_v, sem_g, sem_s):
    c = lax.axis_index("c")
    s = lax.axis_index("s")

    _zero_rows(rows_v, 128)
    _zero_acc(acc, rows_v, s)
    plsc.subcore_barrier()

    base = c * HALF

    def strip(st, _):
        row0 = s * EROWS_PER_TILE + st * STRIP_ROWS
        pltpu.sync_copy(src_hbm.at[pl.ds(row0, STRIP_ROWS)], src_v)
        pltpu.sync_copy(dst_hbm.at[pl.ds(row0, STRIP_ROWS)], dst_v)
        nb = _filter_strip(src_v, dst_v, srcf, ldstf, base)

        @pl.when(nb > 0)
        def _():
            pltpu.async_copy(y_hbm.at[srcf.at[0]],
                             rows_v.at[pl.ds(0, 128)], sem_g)

        def bloop(b, _):
            slot = (b & 1) * 128
            nslot = 128 - slot
            pltpu.make_async_copy(y_hbm.at[srcf.at[b]],
                                  rows_v.at[pl.ds(slot, 128)], sem_g).wait()

            @pl.when(b > 0)
            def _():
                pltpu.make_async_copy(rows_v.at[pl.ds(nslot, 128)],
                                      acc.at[ldstf.at[b - 1]], sem_s).wait()

            pltpu.async_copy(rows_v.at[pl.ds(slot, 128)],
                             acc.at[ldstf.at[b]], sem_s, add=True)

            @pl.when(b + 1 < nb)
            def _():
                pltpu.async_copy(y_hbm.at[srcf.at[b + 1]],
                                 rows_v.at[pl.ds(nslot, 128)], sem_g)

            return 0

        lax.fori_loop(0, nb, bloop, 0)

        @pl.when(nb > 0)
        def _():
            lastslot = ((nb - 1) & 1) * 128
            pltpu.make_async_copy(rows_v.at[pl.ds(lastslot, 128)],
                                  acc.at[ldstf.at[nb - 1]], sem_s).wait()

        return 0

    lax.fori_loop(0, NSTRIPS, strip, 0)
    plsc.subcore_barrier()

    lb = s * ROWS_PER_TILE

    def bstep(k, _):
        r0 = lb + k * BCHUNK
        g0 = c * HALF + r0
        pltpu.sync_copy(acc.at[pl.ds(r0, BCHUNK)], rows_v.at[pl.ds(0, BCHUNK)])
        pltpu.sync_copy(de_hbm.at[pl.ds(g0, BCHUNK)],
                        rows_v.at[pl.ds(64, BCHUNK)])
        pltpu.sync_copy(d2_hbm.at[pl.ds(g0, BCHUNK)],
                        rows_v.at[pl.ds(128, BCHUNK)])
        pltpu.sync_copy(tin_hbm.at[pl.ds(g0, BCHUNK)],
                        rows_v.at[pl.ds(192, BCHUNK)])

        def rowop(i, _):
            for q in range(4):
                sl = pl.ds(q * 16, 16)
                a = rows_v[i, sl]
                rows_v[192 + i, sl] = (rows_v[192 + i, sl]
                                       + 0.25 * (rows_v[64 + i, sl] * a))
                rows_v[i, sl] = rows_v[128 + i, sl] * a
            return 0

        lax.fori_loop(0, BCHUNK, rowop, 0)
        pltpu.sync_copy(rows_v.at[pl.ds(192, BCHUNK)],
                        tout_hbm.at[pl.ds(g0, BCHUNK)])
        pltpu.sync_copy(rows_v.at[pl.ds(0, BCHUNK)],
                        yout_hbm.at[pl.ds(g0, BCHUNK)])
        return 0

    lax.fori_loop(0, BSTEPS, bstep, 0)


def _setup_tc_body(deg_ref, w_ref, de_ref, d2_ref, y0_ref, t0_ref):
    d = deg_ref[:, :]
    dis = jnp.where(d >= 0.5, lax.rsqrt(jnp.maximum(d, 1e-12)), 0.0)
    w = w_ref[:, :]
    de = jnp.broadcast_to(dis, w.shape)
    de_ref[:, :] = de
    d2_ref[:, :] = de * de
    y0_ref[:, :] = de * w
    t0_ref[:, :] = 0.25 * w


_SETUP_BR = 512


def _deg_call(src2d, dst2d):
    f = pl.kernel(
        _deg_body,
        out_type=jax.ShapeDtypeStruct((N_PAD,), jnp.float32),
        mesh=_MESH,
        scratch_types=[
            pltpu.VMEM_SHARED((ACC_ROWS,), jnp.float32),
            pltpu.VMEM((STRIP_ROWS, 128), jnp.int32),
            pltpu.VMEM((STRIP_ROWS, 128), jnp.int32),
            pltpu.VMEM((FROWS, 128), jnp.int32),
            pltpu.VMEM((FROWS, 128), jnp.int32),
            pltpu.VMEM((128,), jnp.float32),
            pltpu.VMEM((ROWS_PER_TILE,), jnp.float32),
            pltpu.SemaphoreType.DMA,
        ],
        compiler_params=pltpu.CompilerParams(use_tc_tiling_on_sc=False, needs_layout_passes=False),
    )
    return f(src2d, dst2d)


def _setup_call(deg, w_pad):
    grid = (N_PAD // _SETUP_BR,)
    bs_deg = pl.BlockSpec((_SETUP_BR, 1), lambda i: (i, 0))
    bs = pl.BlockSpec((_SETUP_BR, D), lambda i: (i, 0))
    out = jax.ShapeDtypeStruct((N_PAD, D), jnp.float32)
    return pl.pallas_call(
        _setup_tc_body,
        grid=grid,
        in_specs=[bs_deg, bs],
        out_specs=[bs, bs, bs, bs],
        out_shape=[out, out, out, out],
    )(deg, w_pad)


def _layer_call(src2d, dst2d, y, de, d2, t):
    out = jax.ShapeDtypeStruct((N_PAD, D), jnp.float32)
    f = pl.kernel(
        _layer_body,
        out_type=(out, out),
        mesh=_MESH,
        scratch_types=[
            pltpu.VMEM_SHARED((ACC_ROWS, D), jnp.float32),
            pltpu.VMEM((STRIP_ROWS, 128), jnp.int32),
            pltpu.VMEM((STRIP_ROWS, 128), jnp.int32),
            pltpu.VMEM((FROWS, 128), jnp.int32),
            pltpu.VMEM((FROWS, 128), jnp.int32),
            pltpu.VMEM((256, D), jnp.float32),
            pltpu.SemaphoreType.DMA,
            pltpu.SemaphoreType.DMA,
        ],
        compiler_params=pltpu.CompilerParams(use_tc_tiling_on_sc=False, needs_layout_passes=False),
    )
    return f(src2d, dst2d, y, de, d2, t)


def kernel(edge_index, weight):
    src = edge_index[0]
    dst = edge_index[1]
    pad = jnp.full((E_PAD - E,), 1 << 29, dtype=jnp.int32)
    src2d = jnp.concatenate([src, jnp.zeros((E_PAD - E,), jnp.int32)]
                            ).reshape(EROWS, 128)
    dst2d = jnp.concatenate([dst, pad]).reshape(EROWS, 128)
    w_pad = jnp.zeros((N_PAD, D), jnp.float32).at[:N_NODES].set(weight)

    deg = _deg_call(src2d, dst2d)
    de, d2, y, t = _setup_call(deg.reshape(N_PAD, 1), w_pad)
    for _ in range(LAYERS):
        y, t = _layer_call(src2d, dst2d, y, de, d2, t)

    return (t[:N_USERS], t[N_USERS:N_NODES])


# speedup vs baseline: 1.1313x; 1.0270x over previous
import jax
import jax.numpy as jnp
from jax import lax
from jax.experimental import pallas as pl
from jax.experimental.pallas import tpu as pltpu
from jax.experimental.pallas import tpu_sc as plsc

N_USERS = 30000
N_ITEMS = 20000
N_NODES = 50000
D = 64
LAYERS = 3
E = 800000

NC, NS = 2, 16
HALF = 25600
N_PAD = NC * HALF
ACC_ROWS = 25728
TRASH = 25600
ROWS_PER_TILE = HALF // NS
BCHUNK = 64
BSTEPS = ROWS_PER_TILE // BCHUNK
E_PAD = 819200
EROWS = E_PAD // 128
EROWS_PER_TILE = EROWS // NS
STRIP_ROWS = 16
NSTRIPS = EROWS_PER_TILE // STRIP_ROWS
FROWS = STRIP_ROWS + 1
ZROWS_T = HALF // NS

_MESH = plsc.VectorSubcoreMesh(core_axis_name="c", subcore_axis_name="s")


def _popcnt(m):
    r = plsc.all_reduce_population_count(m)
    if getattr(r, "ndim", 0):
        r = r[0]
    return r


def _zero_rows(rows_v, n):
    def z0body(i, _):
        for k in range(4):
            rows_v[i, pl.ds(k * 16, 16)] = jnp.zeros((16,), jnp.float32)
        return 0

    lax.fori_loop(0, n, z0body, 0)


def _zero_acc(acc, rows_v, s):
    z0 = s * ZROWS_T

    def zbody(i, _):
        pltpu.sync_copy(rows_v.at[pl.ds(0, 128)],
                        acc.at[pl.ds(z0 + i * 128, 128)])
        return 0

    lax.fori_loop(0, ZROWS_T // 128, zbody, 0)
    rem = ZROWS_T - (ZROWS_T // 128) * 128
    if rem:
        pltpu.sync_copy(rows_v.at[pl.ds(0, rem)],
                        acc.at[pl.ds(z0 + ZROWS_T - rem, rem)])


def _filter_strip(ed_v, srcf, ldstf, base):
    iot = lax.iota(jnp.int32, 16)

    def fbody(i, cur):
        j = i >> 3
        k = i & 7
        sl = pl.ds(k * 16, 16)
        vd = ed_v[j, pl.ds(128 + k * 16, 16)]
        vs = ed_v[j, sl]
        l = vd - base
        m = (l >= 0) & (l < HALF)
        m32 = jnp.where(m, 1, 0)
        rank = plsc.cumsum(m32) - m32
        a = cur + rank
        plsc.store_scatter(srcf, [a >> 7, a & 127], vs, mask=m)
        plsc.store_scatter(ldstf, [a >> 7, a & 127], l, mask=m)
        return cur + _popcnt(m)

    cur = lax.fori_loop(0, STRIP_ROWS * 8, fbody, 0)
    nb = (cur + 127) >> 7
    lim = nb * 128

    def pbody(k, _):
        a = cur + k * 16 + iot
        m = a < lim
        psrc = 50000 + (a & 1023)
        plsc.store_scatter(srcf, [a >> 7, a & 127], psrc, mask=m)
        plsc.store_scatter(ldstf, [a >> 7, a & 127], TRASH + (a & 127),
                           mask=m)
        return 0

    lax.fori_loop(0, 8, pbody, 0)
    return nb


def _deg_body(ed_hbm, deg_hbm, dacc, ed_v, srcf, ldstf,
              ones_v, degb, sem_s):
    c = lax.axis_index("c")
    s = lax.axis_index("s")

    def initv(i, _):
        ones_v[pl.ds(i * 16, 16)] = jnp.ones((16,), jnp.float32)
        degb[pl.ds(i * 16, 16)] = jnp.zeros((16,), jnp.float32)
        return 0

    lax.fori_loop(0, 8, initv, 0)

    z0 = s * ZROWS_T

    def zbody(i, _):
        pltpu.sync_copy(degb.at[pl.ds(0, 128)],
                        dacc.at[pl.ds(z0 + i * 128, 128)])
        return 0

    lax.fori_loop(0, ZROWS_T // 128, zbody, 0)
    rem = ZROWS_T - (ZROWS_T // 128) * 128
    if rem:
        pltpu.sync_copy(degb.at[pl.ds(0, rem)],
                        dacc.at[pl.ds(z0 + ZROWS_T - rem, rem)])
    plsc.subcore_barrier()

    base = c * HALF

    def strip(st, _):
        row0 = s * EROWS_PER_TILE + st * STRIP_ROWS
        pltpu.sync_copy(ed_hbm.at[pl.ds(row0, STRIP_ROWS)], ed_v)
        nb = _filter_strip(ed_v, srcf, ldstf, base)

        def bloop(b, _):
            pltpu.async_copy(ones_v, dacc.at[ldstf.at[b]], sem_s, add=True)
            return 0

        lax.fori_loop(0, nb, bloop, 0)

        def bdrain(b, _):
            pltpu.make_async_copy(ones_v, dacc.at[ldstf.at[b]], sem_s).wait()
            return 0

        lax.fori_loop(0, nb, bdrain, 0)
        return 0

    lax.fori_loop(0, NSTRIPS, strip, 0)
    plsc.subcore_barrier()

    lb = s * ROWS_PER_TILE
    pltpu.sync_copy(dacc.at[pl.ds(lb, ROWS_PER_TILE)], degb)
    pltpu.sync_copy(degb, deg_hbm.at[pl.ds(c * HALF + lb, ROWS_PER_TILE)])


def _layer_body(ed_hbm, y_hbm, de_hbm, d2_hbm, tin_hbm,
                yout_hbm, tout_hbm,
                acc, ed_v, srcf, ldstf, rows_v, sem_g, sem_s):
    c = lax.axis_index("c")
    s = lax.axis_index("s")

    _zero_rows(rows_v, 128)
    _zero_acc(acc, rows_v, s)
    plsc.subcore_barrier()

    base = c * HALF

    def strip(st, _):
        row0 = s * EROWS_PER_TILE + st * STRIP_ROWS
        pltpu.sync_copy(ed_hbm.at[pl.ds(row0, STRIP_ROWS)], ed_v)
        nb = _filter_strip(ed_v, srcf, ldstf, base)

        @pl.when(nb > 0)
        def _():
            pltpu.async_copy(y_hbm.at[srcf.at[0]],
                             rows_v.at[pl.ds(0, 128)], sem_g)

        def bloop(b, _):
            slot = (b & 1) * 128
            nslot = 128 - slot
            pltpu.make_async_copy(y_hbm.at[srcf.at[b]],
                                  rows_v.at[pl.ds(slot, 128)], sem_g).wait()

            @pl.when(b > 0)
            def _():
                pltpu.make_async_copy(rows_v.at[pl.ds(nslot, 128)],
                                      acc.at[ldstf.at[b - 1]], sem_s).wait()

            pltpu.async_copy(rows_v.at[pl.ds(slot, 128)],
                             acc.at[ldstf.at[b]], sem_s, add=True)

            @pl.when(b + 1 < nb)
            def _():
                pltpu.async_copy(y_hbm.at[srcf.at[b + 1]],
                                 rows_v.at[pl.ds(nslot, 128)], sem_g)

            return 0

        lax.fori_loop(0, nb, bloop, 0)

        @pl.when(nb > 0)
        def _():
            lastslot = ((nb - 1) & 1) * 128
            pltpu.make_async_copy(rows_v.at[pl.ds(lastslot, 128)],
                                  acc.at[ldstf.at[nb - 1]], sem_s).wait()

        return 0

    lax.fori_loop(0, NSTRIPS, strip, 0)
    plsc.subcore_barrier()

    lb = s * ROWS_PER_TILE

    def bstep(k, _):
        r0 = lb + k * BCHUNK
        g0 = c * HALF + r0
        pltpu.sync_copy(acc.at[pl.ds(r0, BCHUNK)], rows_v.at[pl.ds(0, BCHUNK)])
        pltpu.sync_copy(de_hbm.at[pl.ds(g0, BCHUNK)],
                        rows_v.at[pl.ds(64, BCHUNK)])
        pltpu.sync_copy(d2_hbm.at[pl.ds(g0, BCHUNK)],
                        rows_v.at[pl.ds(128, BCHUNK)])
        pltpu.sync_copy(tin_hbm.at[pl.ds(g0, BCHUNK)],
                        rows_v.at[pl.ds(192, BCHUNK)])

        def rowop(i, _):
            for q in range(4):
                sl = pl.ds(q * 16, 16)
                a = rows_v[i, sl]
                rows_v[192 + i, sl] = (rows_v[192 + i, sl]
                                       + 0.25 * (rows_v[64 + i, sl] * a))
                rows_v[i, sl] = rows_v[128 + i, sl] * a
            return 0

        lax.fori_loop(0, BCHUNK, rowop, 0)
        pltpu.sync_copy(rows_v.at[pl.ds(192, BCHUNK)],
                        tout_hbm.at[pl.ds(g0, BCHUNK)])
        pltpu.sync_copy(rows_v.at[pl.ds(0, BCHUNK)],
                        yout_hbm.at[pl.ds(g0, BCHUNK)])
        return 0

    lax.fori_loop(0, BSTEPS, bstep, 0)


def _setup_tc_body(deg_ref, w_ref, de_ref, d2_ref, y0_ref, t0_ref):
    d = deg_ref[:, :]
    dis = jnp.where(d >= 0.5, lax.rsqrt(jnp.maximum(d, 1e-12)), 0.0)
    w = w_ref[:, :]
    de = jnp.broadcast_to(dis, w.shape)
    de_ref[:, :] = de
    d2_ref[:, :] = de * de
    y0_ref[:, :] = de * w
    t0_ref[:, :] = 0.25 * w


_SETUP_BR = 512


def _deg_call(ed2d):
    f = pl.kernel(
        _deg_body,
        out_type=jax.ShapeDtypeStruct((N_PAD,), jnp.float32),
        mesh=_MESH,
        scratch_types=[
            pltpu.VMEM_SHARED((ACC_ROWS,), jnp.float32),
            pltpu.VMEM((STRIP_ROWS, 256), jnp.int32),
            pltpu.VMEM((FROWS, 128), jnp.int32),
            pltpu.VMEM((FROWS, 128), jnp.int32),
            pltpu.VMEM((128,), jnp.float32),
            pltpu.VMEM((ROWS_PER_TILE,), jnp.float32),
            pltpu.SemaphoreType.DMA,
        ],
        compiler_params=pltpu.CompilerParams(use_tc_tiling_on_sc=False, needs_layout_passes=False),
    )
    return f(ed2d)


def _setup_call(deg, w_pad):
    grid = (N_PAD // _SETUP_BR,)
    bs_deg = pl.BlockSpec((_SETUP_BR, 1), lambda i: (i, 0))
    bs = pl.BlockSpec((_SETUP_BR, D), lambda i: (i, 0))
    out = jax.ShapeDtypeStruct((N_PAD, D), jnp.float32)
    return pl.pallas_call(
        _setup_tc_body,
        grid=grid,
        in_specs=[bs_deg, bs],
        out_specs=[bs, bs, bs, bs],
        out_shape=[out, out, out, out],
    )(deg, w_pad)


def _layer_call(ed2d, y, de, d2, t):
    out = jax.ShapeDtypeStruct((N_PAD, D), jnp.float32)
    f = pl.kernel(
        _layer_body,
        out_type=(out, out),
        mesh=_MESH,
        scratch_types=[
            pltpu.VMEM_SHARED((ACC_ROWS, D), jnp.float32),
            pltpu.VMEM((STRIP_ROWS, 256), jnp.int32),
            pltpu.VMEM((FROWS, 128), jnp.int32),
            pltpu.VMEM((FROWS, 128), jnp.int32),
            pltpu.VMEM((256, D), jnp.float32),
            pltpu.SemaphoreType.DMA,
            pltpu.SemaphoreType.DMA,
        ],
        compiler_params=pltpu.CompilerParams(use_tc_tiling_on_sc=False, needs_layout_passes=False),
    )
    return f(ed2d, y, de, d2, t)


def kernel(edge_index, weight):
    src = edge_index[0]
    dst = edge_index[1]
    pad = jnp.full((E_PAD - E,), 1 << 29, dtype=jnp.int32)
    src2d = jnp.concatenate([src, jnp.zeros((E_PAD - E,), jnp.int32)]
                            ).reshape(EROWS, 128)
    dst2d = jnp.concatenate([dst, pad]).reshape(EROWS, 128)
    ed2d = jnp.concatenate([src2d, dst2d], axis=1)
    w_pad = jnp.zeros((N_PAD, D), jnp.float32).at[:N_NODES].set(weight)

    deg = _deg_call(ed2d)
    de, d2, y, t = _setup_call(deg.reshape(N_PAD, 1), w_pad)
    for _ in range(LAYERS):
        y, t = _layer_call(ed2d, y, de, d2, t)

    return (t[:N_USERS], t[N_USERS:N_NODES])
